# baseline (speedup 1.0000x reference)
import functools

import jax
import jax.numpy as jnp
from jax.experimental import pallas as pl
from jax.experimental.pallas import tpu as pltpu

DIM = 4096
BZ = 4
SEQ = 4096
NUM_EXPERTS = 64
R = 8
OUT_LANES = 128

SLOT_ROWS = 896
NSLOT = 2

_RAMP = [64, 64, 128, 256, 896, 896, 896, 896]
_STEADY = [896, 896, 896, 896, 512]


def _stream_chunks(s):
    chunks = []
    for bb, sizes in ((2 * s, _RAMP), (2 * s + 1, _STEADY)):
        r0 = 0
        for sz in sizes:
            chunks.append((bb, r0, sz, r0 + sz == SEQ))
            r0 += sz
    return chunks


_S = [_stream_chunks(0), _stream_chunks(1)]
NC = len(_S[0])
assert len(_S[1]) == NC


def _router_kernel(x_hbm, wa_ref, wb_ref, outa_ref, outb_ref,
                   buf_ref, lg_ref, sems):

    def chunk_copy(s, i):
        b, r0, rows, _ = _S[s][i]
        slot = s * NSLOT + i % NSLOT
        return pltpu.make_async_copy(
            x_hbm.at[b, pl.ds(r0, rows), :],
            buf_ref.at[slot, pl.ds(0, rows), :],
            sems.at[slot])

    def gate_logits(acc):
        s4 = acc[0:4, :] + acc[4:8, :]
        s2 = s4[0:2, :] + s4[2:4, :]
        s1 = s2[0:1, :] + s2[1:2, :]
        mean = s1 * (1.0 / SEQ)
        la = jax.lax.dot_general(
            mean, wa_ref[...],
            dimension_numbers=(((1,), (1,)), ((), ())),
            preferred_element_type=jnp.float32,
        )
        lb = jax.lax.dot_general(
            mean, wb_ref[...],
            dimension_numbers=(((1,), (1,)), ((), ())),
            preferred_element_type=jnp.float32,
        )
        return la, lb

    for i in range(NSLOT):
        chunk_copy(0, i).start()
        chunk_copy(1, i).start()

    acc = [None, None]
    for i in range(NC):
        for s in range(2):
            b, r0, rows, is_last = _S[s][i]
            slot = s * NSLOT + i % NSLOT
            chunk_copy(s, i).wait()
            a = acc[s]
            for k in range(rows // 8):
                g = buf_ref[slot, 8 * k:8 * k + 8, :]
                a = g if a is None else a + g
            acc[s] = a
            if i + NSLOT < NC:
                chunk_copy(s, i + NSLOT).start()
            if is_last:
                la, lb = gate_logits(acc[s])
                lg_ref[b:b + 1, 0:NUM_EXPERTS] = la
                lg_ref[b:b + 1, NUM_EXPERTS:] = lb
                acc[s] = None

    lg = lg_ref[0:BZ, :]

    def topk_rows(vals):
        lanes = jax.lax.broadcasted_iota(jnp.int32, (1, NUM_EXPERTS), 1)
        out_lanes = jax.lax.broadcasted_iota(jnp.int32, (1, OUT_LANES), 1)
        rows = jnp.zeros((BZ, OUT_LANES), dtype=jnp.int32)
        for i in range(R):
            m = jnp.max(vals, axis=1, keepdims=True)
            cand = jnp.where(vals == m, lanes, NUM_EXPERTS)
            idx = jnp.min(cand, axis=1, keepdims=True)
            rows = jnp.where(out_lanes == i, idx, rows)
            vals = jnp.where(lanes == idx, -jnp.inf, vals)
        return rows

    outa_ref[...] = topk_rows(lg[:, 0:NUM_EXPERTS])[:, 0:R]
    outb_ref[...] = topk_rows(lg[:, NUM_EXPERTS:])[:, 0:R]


@functools.partial(jax.jit, static_argnames=("interpret",))
def kernel(input_x, WA, WB, interpret=False):
    out_shape = jax.ShapeDtypeStruct((BZ, R), jnp.int32)
    return pl.pallas_call(
        _router_kernel,
        in_specs=[
            pl.BlockSpec(memory_space=pltpu.MemorySpace.HBM),
            pl.BlockSpec(memory_space=pltpu.MemorySpace.VMEM),
            pl.BlockSpec(memory_space=pltpu.MemorySpace.VMEM),
        ],
        out_specs=[
            pl.BlockSpec(memory_space=pltpu.MemorySpace.VMEM),
            pl.BlockSpec(memory_space=pltpu.MemorySpace.VMEM),
        ],
        out_shape=[out_shape, out_shape],
        scratch_shapes=[
            pltpu.VMEM((2 * NSLOT, SLOT_ROWS, DIM), jnp.float32),
            pltpu.VMEM((8, 2 * NUM_EXPERTS), jnp.float32),
            pltpu.SemaphoreType.DMA((2 * NSLOT,)),
        ],
        interpret=interpret,
    )(input_x, WA, WB)


# speedup vs baseline: 1.0088x; 1.0088x over previous
import functools

import jax
import jax.numpy as jnp
from jax.experimental import pallas as pl
from jax.experimental.pallas import tpu as pltpu

DIM = 4096
BZ = 4
SEQ = 4096
NUM_EXPERTS = 64
R = 8
OUT_LANES = 128

SLOT_ROWS = 512
NSLOT = 3

_RAMP = [64, 64, 128, 256, 512, 512, 512, 512, 512, 512, 512]
_STEADY = [SLOT_ROWS] * (SEQ // SLOT_ROWS)


def _stream_chunks(s):
    chunks = []
    for bb, sizes in ((2 * s, _RAMP), (2 * s + 1, _STEADY)):
        r0 = 0
        for sz in sizes:
            chunks.append((bb, r0, sz, r0 + sz == SEQ))
            r0 += sz
    return chunks


_S = [_stream_chunks(0), _stream_chunks(1)]
NC = len(_S[0])
assert len(_S[1]) == NC


def _router_kernel(x_hbm, wa_ref, wb_ref, outa_ref, outb_ref,
                   buf_ref, lg_ref, sems):

    def chunk_copy(s, i):
        b, r0, rows, _ = _S[s][i]
        slot = s * NSLOT + i % NSLOT
        return pltpu.make_async_copy(
            x_hbm.at[b, pl.ds(r0, rows), :],
            buf_ref.at[slot, pl.ds(0, rows), :],
            sems.at[slot])

    def gate_logits(acc):
        s4 = acc[0:4, :] + acc[4:8, :]
        s2 = s4[0:2, :] + s4[2:4, :]
        s1 = s2[0:1, :] + s2[1:2, :]
        mean = s1 * (1.0 / SEQ)
        la = jax.lax.dot_general(
            mean, wa_ref[...],
            dimension_numbers=(((1,), (1,)), ((), ())),
            preferred_element_type=jnp.float32,
        )
        lb = jax.lax.dot_general(
            mean, wb_ref[...],
            dimension_numbers=(((1,), (1,)), ((), ())),
            preferred_element_type=jnp.float32,
        )
        return la, lb

    for i in range(NSLOT):
        chunk_copy(0, i).start()
        chunk_copy(1, i).start()

    acc = [None, None]
    for i in range(NC):
        for s in range(2):
            b, r0, rows, is_last = _S[s][i]
            slot = s * NSLOT + i % NSLOT
            chunk_copy(s, i).wait()
            a = acc[s]
            for k in range(rows // 8):
                g = buf_ref[slot, 8 * k:8 * k + 8, :]
                a = g if a is None else a + g
            acc[s] = a
            if i + NSLOT < NC:
                chunk_copy(s, i + NSLOT).start()
            if is_last:
                la, lb = gate_logits(acc[s])
                lg_ref[b:b + 1, 0:NUM_EXPERTS] = la
                lg_ref[b:b + 1, NUM_EXPERTS:] = lb
                acc[s] = None

    lg = lg_ref[0:BZ, :]

    def topk_rows(vals):
        lanes = jax.lax.broadcasted_iota(jnp.int32, (1, NUM_EXPERTS), 1)
        out_lanes = jax.lax.broadcasted_iota(jnp.int32, (1, OUT_LANES), 1)
        rows = jnp.zeros((BZ, OUT_LANES), dtype=jnp.int32)
        for i in range(R):
            m = jnp.max(vals, axis=1, keepdims=True)
            cand = jnp.where(vals == m, lanes, NUM_EXPERTS)
            idx = jnp.min(cand, axis=1, keepdims=True)
            rows = jnp.where(out_lanes == i, idx, rows)
            vals = jnp.where(lanes == idx, -jnp.inf, vals)
        return rows

    outa_ref[...] = topk_rows(lg[:, 0:NUM_EXPERTS])[:, 0:R]
    outb_ref[...] = topk_rows(lg[:, NUM_EXPERTS:])[:, 0:R]


@functools.partial(jax.jit, static_argnames=("interpret",))
def kernel(input_x, WA, WB, interpret=False):
    out_shape = jax.ShapeDtypeStruct((BZ, R), jnp.int32)
    return pl.pallas_call(
        _router_kernel,
        in_specs=[
            pl.BlockSpec(memory_space=pltpu.MemorySpace.HBM),
            pl.BlockSpec(memory_space=pltpu.MemorySpace.VMEM),
            pl.BlockSpec(memory_space=pltpu.MemorySpace.VMEM),
        ],
        out_specs=[
            pl.BlockSpec(memory_space=pltpu.MemorySpace.VMEM),
            pl.BlockSpec(memory_space=pltpu.MemorySpace.VMEM),
        ],
        out_shape=[out_shape, out_shape],
        scratch_shapes=[
            pltpu.VMEM((2 * NSLOT, SLOT_ROWS, DIM), jnp.float32),
            pltpu.VMEM((8, 2 * NUM_EXPERTS), jnp.float32),
            pltpu.SemaphoreType.DMA((2 * NSLOT,)),
        ],
        interpret=interpret,
    )(input_x, WA, WB)
